# baseline (speedup 1.0000x reference)
import functools

import jax
import jax.numpy as jnp
from jax import lax
from jax.experimental import pallas as pl
from jax.experimental.pallas import tpu as pltpu
from jax.experimental.pallas import tpu_sc as plsc

NUM_USERS = 1000000
EMBED_DIM = 64
BATCH = 16384

_info = plsc.get_sparse_core_info()
_NC = _info.num_cores
_NS = _info.num_subcores
_NW = _NC * _NS
_BW = BATCH // _NW
_E = _BW * EMBED_DIM


@functools.partial(
    pl.kernel,
    mesh=plsc.VectorSubcoreMesh(core_axis_name="c", subcore_axis_name="s"),
    out_type=jax.ShapeDtypeStruct((BATCH * EMBED_DIM,), jnp.float32),
    scratch_types=[
        pltpu.VMEM((_BW,), jnp.int32),
        pltpu.VMEM((_E,), jnp.int32),
        pltpu.VMEM((_E,), jnp.float32),
        pltpu.SemaphoreType.DMA,
        pltpu.SemaphoreType.DMA,
    ],
    compiler_params=pltpu.CompilerParams(use_tc_tiling_on_sc=False),
)
def _sc_gather(idx_hbm, flat_table_hbm, out_hbm, idx_v, fe_v, val_v, gsem, osem):
    wid = lax.axis_index("s") * _NC + lax.axis_index("c")
    base = wid * _BW
    pltpu.sync_copy(idx_hbm.at[pl.ds(base, _BW)], idx_v)

    def dim_body(d, _):
        off = d * NUM_USERS

        def chunk_body(k, _):
            iv = idx_v[pl.ds(k * 16, 16)]
            fe_v[pl.ds(d * _BW + k * 16, 16)] = iv + off
            return 0

        return lax.fori_loop(0, _BW // 16, chunk_body, 0)

    lax.fori_loop(0, EMBED_DIM, dim_body, 0)

    pltpu.async_copy(flat_table_hbm.at[fe_v], val_v, gsem).wait()

    handles = []
    for d in range(EMBED_DIM):
        handles.append(
            pltpu.async_copy(
                val_v.at[pl.ds(d * _BW, _BW)],
                out_hbm.at[pl.ds(d * BATCH + base, _BW)],
                osem,
            )
        )
    for h in handles:
        h.wait()


def kernel(user_inputs, table):
    flat_table = jnp.reshape(jnp.transpose(table), (NUM_USERS * EMBED_DIM,))
    out_flat = _sc_gather(user_inputs, flat_table)
    return jnp.reshape(out_flat, (EMBED_DIM, BATCH)).T


# speedup vs baseline: 19.8547x; 19.8547x over previous
import functools

import jax
import jax.numpy as jnp
from jax import lax
from jax.experimental import pallas as pl
from jax.experimental.pallas import tpu as pltpu
from jax.experimental.pallas import tpu_sc as plsc

NUM_USERS = 1000000
EMBED_DIM = 64
BATCH = 16384

_info = plsc.get_sparse_core_info()
_NC = _info.num_cores
_NS = _info.num_subcores
_NW = _NC * _NS
_BW = BATCH // _NW
_NSLOT = 4

_LAST_WIN = (NUM_USERS // 128 - 1) * 128
_TAIL = (NUM_USERS // 128) * 128


@functools.partial(
    pl.kernel,
    mesh=plsc.VectorSubcoreMesh(core_axis_name="c", subcore_axis_name="s"),
    out_type=jax.ShapeDtypeStruct((BATCH * EMBED_DIM,), jnp.float32),
    scratch_types=[
        pltpu.VMEM((_BW,), jnp.int32),
        [pltpu.VMEM((EMBED_DIM, 128), jnp.float32) for _ in range(_NSLOT)],
        [pltpu.VMEM((EMBED_DIM,), jnp.float32) for _ in range(_NSLOT)],
        pltpu.VMEM((EMBED_DIM, EMBED_DIM), jnp.float32),
        [pltpu.SemaphoreType.DMA for _ in range(_NSLOT)],
        pltpu.SemaphoreType.DMA,
    ],
    compiler_params=pltpu.CompilerParams(needs_layout_passes=False),
)
def _sc_gather(idx_hbm, tt_hbm, tail_hbm, out_hbm,
               idx_v, bufs, orows, tail_v, gsems, osem):
    wid = lax.axis_index("s") * _NC + lax.axis_index("c")
    base = wid * _BW
    pltpu.sync_copy(idx_hbm.at[pl.ds(base, _BW)], idx_v)
    pltpu.sync_copy(tail_hbm, tail_v)

    lanes = lax.iota(jnp.int32, 16)

    def read_idx(i):
        iv = idx_v[pl.ds(pl.multiple_of((i >> 4) << 4, 16), 16)]
        sel = jnp.where(lanes == (i & 15), iv, 0)
        return jnp.sum(sel)

    def win_start(u):
        return pl.multiple_of(
            jnp.minimum((u >> 7) * 128, _LAST_WIN), 128
        )

    def issue(i, slot):
        u = read_idx(i)
        pltpu.async_copy(
            tt_hbm.at[:, pl.ds(win_start(u), 128)], bufs[slot], gsems[slot]
        )

    def extract(i, slot):
        u = read_idx(i)
        pltpu.make_async_copy(
            tt_hbm.at[:, pl.ds(0, 128)], bufs[slot], gsems[slot]
        ).wait()
        lane = jnp.minimum(u - win_start(u), 127)
        tl = jnp.minimum(jnp.maximum(u - _TAIL, 0), EMBED_DIM - 1)
        is_tail = (u >= _TAIL).astype(jnp.int32)
        tail_m = (jnp.zeros((16,), jnp.int32) + is_tail) != 0
        lane_v = jnp.zeros((16,), jnp.int32) + lane
        tl_v = jnp.zeros((16,), jnp.int32) + tl
        for k in range(EMBED_DIM // 16):
            dvec = lanes + (k * 16)
            g1 = plsc.load_gather(bufs[slot], [dvec, lane_v])
            g2 = plsc.load_gather(tail_v, [tl_v, dvec])
            orows[slot][pl.ds(k * 16, 16)] = jnp.where(tail_m, g2, g1)
        pltpu.async_copy(
            orows[slot],
            out_hbm.at[pl.ds((base + i) * EMBED_DIM, EMBED_DIM)],
            osem,
        )

    def drain_orow(slot):
        pltpu.make_async_copy(
            tt_hbm.at[0, pl.ds(0, EMBED_DIM)], orows[slot], osem
        ).wait()

    for s in range(_NSLOT):
        issue(s, s)

    def body(t, _):
        for s in range(_NSLOT):
            i = t * _NSLOT + s

            @pl.when(t > 0)
            def _():
                drain_orow(s)

            extract(i, s)

            @pl.when(i + _NSLOT < _BW)
            def _():
                issue(i + _NSLOT, s)
        return 0

    lax.fori_loop(0, _BW // _NSLOT, body, 0)
    for s in range(_NSLOT):
        drain_orow(s)


def kernel(user_inputs, table):
    tt = jnp.transpose(table)
    tail = table[_TAIL:, :]
    out_flat = _sc_gather(user_inputs, tt, tail)
    return jnp.reshape(out_flat, (BATCH, EMBED_DIM))


# speedup vs baseline: 26.3911x; 1.3292x over previous
import functools

import jax
import jax.numpy as jnp
from jax import lax
from jax.experimental import pallas as pl
from jax.experimental.pallas import tpu as pltpu
from jax.experimental.pallas import tpu_sc as plsc

NUM_USERS = 1000000
EMBED_DIM = 64
BATCH = 16384

_info = plsc.get_sparse_core_info()
_NC = _info.num_cores
_NS = _info.num_subcores
_NW = _NC * _NS

_CW = 256
_TAIL = (NUM_USERS // 128) * 128
_NCH = _TAIL // _CW
_CPW = _NCH // _NW
_XTRA = _NCH - _CPW * _NW
_ORING = 32


@functools.partial(
    pl.kernel,
    mesh=plsc.VectorSubcoreMesh(core_axis_name="c", subcore_axis_name="s"),
    out_type=jax.ShapeDtypeStruct((BATCH * EMBED_DIM,), jnp.float32),
    scratch_types=[
        pltpu.VMEM((BATCH,), jnp.int32),
        pltpu.VMEM((BATCH + 16,), jnp.int32),
        pltpu.VMEM((BATCH + 16,), jnp.int32),
        pltpu.VMEM((BATCH + 16,), jnp.int32),
        pltpu.VMEM((BATCH + 16,), jnp.int32),
        pltpu.VMEM((EMBED_DIM, _CW), jnp.float32),
        pltpu.VMEM((EMBED_DIM, _CW), jnp.float32),
        pltpu.VMEM((EMBED_DIM, EMBED_DIM), jnp.float32),
        pltpu.VMEM((_ORING * EMBED_DIM,), jnp.float32),
        pltpu.SMEM((128,), jnp.int32),
        pltpu.SMEM((128,), jnp.int32),
        pltpu.SMEM((128,), jnp.int32),
        pltpu.SMEM((8,), jnp.int32),
        pltpu.SemaphoreType.DMA,
        pltpu.SemaphoreType.DMA,
        pltpu.SemaphoreType.DMA,
    ],
    compiler_params=pltpu.CompilerParams(needs_layout_passes=False),
)
def _sc_gather(idx_hbm, tt_hbm, tail_hbm, out_hbm,
               idx_all, pu, pg, su, sg, buf0, buf1, tail_v, oring,
               counts, starts, cursor, scal,
               gsem0, gsem1, osem):
    wid = lax.axis_index("s") * _NC + lax.axis_index("c")
    base_c = wid * _CPW + jnp.minimum(wid, _XTRA)
    n_c = _CPW + (wid < _XTRA).astype(jnp.int32)

    lanes = lax.iota(jnp.int32, 16)
    z16 = jnp.zeros((16,), jnp.int32)
    lane0 = lanes == 0

    def issue(c, buf, sem):
        start = pl.multiple_of((base_c + c) * _CW, 128)
        pltpu.async_copy(tt_hbm.at[:, pl.ds(start, _CW)], buf, sem)

    def drain(buf, sem):
        pltpu.make_async_copy(tt_hbm.at[:, pl.ds(0, _CW)], buf, sem).wait()

    pltpu.sync_copy(idx_hbm, idx_all)
    issue(0, buf0, gsem0)

    @pl.when(n_c > 1)
    def _():
        issue(1, buf1, gsem1)

    pltpu.sync_copy(tail_hbm, tail_v)

    def p1_body(v, off):
        iv = idx_all[pl.ds(v * 16, 16)]
        cu = jnp.minimum(iv, _TAIL - 1)
        c = cu >> 8
        m = (c >= base_c) & (c < base_c + n_c)
        mi = m.astype(jnp.int32)
        cnt = jnp.sum(mi)
        pos = jnp.minimum(off + plsc.cumsum(mi) - 1, BATCH - 1)
        plsc.store_scatter(pu, [pos], iv, mask=m)
        plsc.store_scatter(pg, [pos], v * 16 + lanes, mask=m)
        return off + cnt

    off = lax.fori_loop(0, BATCH // 16, p1_body, 0)

    def zinit(j, _):
        counts[j] = 0
        return 0

    lax.fori_loop(0, 128, zinit, 0)

    def p2_body(p, _):
        u = pu[pl.ds(p, 16)][0]
        c_l = (jnp.minimum(u, _TAIL - 1) >> 8) - base_c
        counts[c_l] = counts[c_l] + 1
        return 0

    lax.fori_loop(0, off, p2_body, 0)

    def p3_body(j, run):
        starts[j] = run
        cursor[j] = run
        return run + counts[j]

    lax.fori_loop(0, 128, p3_body, 0)

    def p4_body(p, _):
        u = pu[pl.ds(p, 16)][0]
        g = pg[pl.ds(p, 16)][0]
        c_l = (jnp.minimum(u, _TAIL - 1) >> 8) - base_c
        pos = cursor[c_l]
        cursor[c_l] = pos + 1
        plsc.store_scatter(su, [z16 + pos], z16 + u, mask=lane0)
        plsc.store_scatter(sg, [z16 + pos], z16 + g, mask=lane0)
        return 0

    lax.fori_loop(0, off, p4_body, 0)

    scal[0] = 0

    def extract_chunk(c, buf):
        cbase = (base_c + c) * _CW
        s0 = starts[c]
        cnt = counts[c]

        def ubody(p, _):
            u = su[pl.ds(p, 16)][0]
            g = sg[pl.ds(p, 16)][0]
            q = scal[0]
            scal[0] = q + 1
            slot = q & (_ORING - 1)
            obase = pl.multiple_of(slot * EMBED_DIM, EMBED_DIM)

            @pl.when(q >= _ORING)
            def _():
                pltpu.make_async_copy(
                    out_hbm.at[pl.ds(0, EMBED_DIM)],
                    oring.at[pl.ds(0, EMBED_DIM)],
                    osem,
                ).wait()

            lane = jnp.minimum(u - cbase, _CW - 1)
            is_tail = (u >= _TAIL).astype(jnp.int32)
            tail_m = (z16 + is_tail) != 0
            lane_v = z16 + lane
            tl_v = z16 + jnp.minimum(jnp.maximum(u - _TAIL, 0), EMBED_DIM - 1)
            for k in range(EMBED_DIM // 16):
                dvec = lanes + (k * 16)
                g1 = plsc.load_gather(buf, [dvec, lane_v])
                g2 = plsc.load_gather(tail_v, [tl_v, dvec])
                oring[pl.ds(obase + k * 16, 16)] = jnp.where(tail_m, g2, g1)
            pltpu.async_copy(
                oring.at[pl.ds(obase, EMBED_DIM)],
                out_hbm.at[pl.ds(g * EMBED_DIM, EMBED_DIM)],
                osem,
            )
            return 0

        lax.fori_loop(s0, s0 + cnt, ubody, 0)

    def pair_body(t, _):
        e = 2 * t

        drain(buf0, gsem0)
        extract_chunk(e, buf0)

        @pl.when(e + 2 < n_c)
        def _():
            issue(e + 2, buf0, gsem0)

        @pl.when(e + 1 < n_c)
        def _():
            drain(buf1, gsem1)
            extract_chunk(e + 1, buf1)

            @pl.when(e + 3 < n_c)
            def _():
                issue(e + 3, buf1, gsem1)

        return 0

    lax.fori_loop(0, (n_c + 1) // 2, pair_body, 0)

    def odrain(r, _):
        pltpu.make_async_copy(
            out_hbm.at[pl.ds(0, EMBED_DIM)],
            oring.at[pl.ds(0, EMBED_DIM)],
            osem,
        ).wait()
        return 0

    lax.fori_loop(0, jnp.minimum(off, _ORING), odrain, 0)


def kernel(user_inputs, table):
    tt = jnp.transpose(table)
    tail = table[_TAIL:, :]
    out_flat = _sc_gather(user_inputs, tt, tail)
    return jnp.reshape(out_flat, (BATCH, EMBED_DIM))


# speedup vs baseline: 28.1820x; 1.0679x over previous
import functools

import jax
import jax.numpy as jnp
from jax import lax
from jax.experimental import pallas as pl
from jax.experimental.pallas import tpu as pltpu
from jax.experimental.pallas import tpu_sc as plsc

NUM_USERS = 1000000
EMBED_DIM = 64
BATCH = 16384

_info = plsc.get_sparse_core_info()
_NC = _info.num_cores
_NS = _info.num_subcores
_NW = _NC * _NS

_CW = 128
_TAIL = (NUM_USERS // 128) * 128
_NCH = _TAIL // _CW
_CPW = _NCH // _NW
_XTRA = _NCH - _CPW * _NW
_NBUF = 4
_ORING = 32
_CSH = 7


@functools.partial(
    pl.kernel,
    mesh=plsc.VectorSubcoreMesh(core_axis_name="c", subcore_axis_name="s"),
    out_type=jax.ShapeDtypeStruct((BATCH * EMBED_DIM,), jnp.float32),
    scratch_types=[
        pltpu.VMEM((BATCH,), jnp.int32),
        pltpu.VMEM((BATCH + 16,), jnp.int32),
        pltpu.VMEM((BATCH + 16,), jnp.int32),
        pltpu.VMEM((BATCH + 16,), jnp.int32),
        pltpu.VMEM((BATCH + 16,), jnp.int32),
        [pltpu.VMEM((EMBED_DIM, _CW), jnp.float32) for _ in range(_NBUF)],
        pltpu.VMEM((EMBED_DIM, EMBED_DIM), jnp.float32),
        pltpu.VMEM((_ORING * EMBED_DIM,), jnp.float32),
        pltpu.SMEM((256,), jnp.int32),
        pltpu.SMEM((256,), jnp.int32),
        pltpu.SMEM((256,), jnp.int32),
        pltpu.SMEM((8,), jnp.int32),
        [pltpu.SemaphoreType.DMA for _ in range(_NBUF)],
        pltpu.SemaphoreType.DMA,
    ],
    compiler_params=pltpu.CompilerParams(needs_layout_passes=False),
)
def _sc_gather(idx_hbm, tt_hbm, tail_hbm, out_hbm,
               idx_all, pu, pg, su, sg, bufs, tail_v, oring,
               counts, starts, cursor, scal,
               gsems, osem):
    wid = lax.axis_index("s") * _NC + lax.axis_index("c")
    base_c = wid * _CPW + jnp.minimum(wid, _XTRA)
    n_c = _CPW + (wid < _XTRA).astype(jnp.int32)

    lanes = lax.iota(jnp.int32, 16)
    z16 = jnp.zeros((16,), jnp.int32)
    lane0 = lanes == 0

    pltpu.sync_copy(idx_hbm, idx_all)
    pltpu.sync_copy(tail_hbm, tail_v)

    def p1_body(v, off):
        iv = idx_all[pl.ds(v * 16, 16)]
        cu = jnp.minimum(iv, _TAIL - 1)
        c = cu >> _CSH
        m = (c >= base_c) & (c < base_c + n_c)
        mi = m.astype(jnp.int32)
        cnt = jnp.sum(mi)

        @pl.when(cnt > 0)
        def _():
            pos = jnp.minimum(off + plsc.cumsum(mi) - 1, BATCH - 1)
            plsc.store_scatter(pu, [pos], iv, mask=m)
            plsc.store_scatter(pg, [pos], v * 16 + lanes, mask=m)

        return off + cnt

    off = lax.fori_loop(0, BATCH // 16, p1_body, 0)

    def zinit(j, _):
        counts[j] = 0
        return 0

    lax.fori_loop(0, 256, zinit, 0)

    def p2_body(p, _):
        u = pu[pl.ds(p, 16)][0]
        c_l = (jnp.minimum(u, _TAIL - 1) >> _CSH) - base_c
        counts[c_l] = counts[c_l] + 1
        return 0

    lax.fori_loop(0, off, p2_body, 0)

    def p3_body(j, run):
        starts[j] = run
        cursor[j] = run
        return run + counts[j]

    lax.fori_loop(0, 256, p3_body, 0)

    def p4_body(p, _):
        u = pu[pl.ds(p, 16)][0]
        g = pg[pl.ds(p, 16)][0]
        c_l = (jnp.minimum(u, _TAIL - 1) >> _CSH) - base_c
        pos = cursor[c_l]
        cursor[c_l] = pos + 1
        plsc.store_scatter(su, [z16 + pos], z16 + u, mask=lane0)
        plsc.store_scatter(sg, [z16 + pos], z16 + g, mask=lane0)
        return 0

    lax.fori_loop(0, off, p4_body, 0)

    scal[0] = 0

    def hit(c):
        return (c < n_c) & (counts[c] > 0)

    def issue(c, s):
        start = pl.multiple_of((base_c + c) * _CW, 128)
        pltpu.async_copy(tt_hbm.at[:, pl.ds(start, _CW)], bufs[s], gsems[s])

    def drain(s):
        pltpu.make_async_copy(
            tt_hbm.at[:, pl.ds(0, _CW)], bufs[s], gsems[s]
        ).wait()

    def extract_chunk(c, s):
        cbase = (base_c + c) * _CW
        s0 = starts[c]
        cnt = counts[c]

        def ubody(p, _):
            u = su[pl.ds(p, 16)][0]
            g = sg[pl.ds(p, 16)][0]
            q = scal[0]
            scal[0] = q + 1
            slot = q & (_ORING - 1)
            obase = pl.multiple_of(slot * EMBED_DIM, EMBED_DIM)

            @pl.when(q >= _ORING)
            def _():
                pltpu.make_async_copy(
                    out_hbm.at[pl.ds(0, EMBED_DIM)],
                    oring.at[pl.ds(0, EMBED_DIM)],
                    osem,
                ).wait()

            lane = jnp.minimum(u - cbase, _CW - 1)
            is_tail = (u >= _TAIL).astype(jnp.int32)
            tail_m = (z16 + is_tail) != 0
            lane_v = z16 + lane
            tl_v = z16 + jnp.minimum(jnp.maximum(u - _TAIL, 0), EMBED_DIM - 1)
            for k in range(EMBED_DIM // 16):
                dvec = lanes + (k * 16)
                g1 = plsc.load_gather(bufs[s], [dvec, lane_v])
                g2 = plsc.load_gather(tail_v, [tl_v, dvec])
                oring[pl.ds(obase + k * 16, 16)] = jnp.where(tail_m, g2, g1)
            pltpu.async_copy(
                oring.at[pl.ds(obase, EMBED_DIM)],
                out_hbm.at[pl.ds(g * EMBED_DIM, EMBED_DIM)],
                osem,
            )
            return 0

        lax.fori_loop(s0, s0 + cnt, ubody, 0)

    for s in range(_NBUF):
        @pl.when(hit(s))
        def _(s=s):
            issue(s, s)

    def ring_body(t, _):
        for s in range(_NBUF):
            c = t * _NBUF + s

            @pl.when(hit(c))
            def _(c=c, s=s):
                drain(s)
                extract_chunk(c, s)

            @pl.when(hit(c + _NBUF))
            def _(c=c, s=s):
                issue(c + _NBUF, s)
        return 0

    lax.fori_loop(0, (_CPW + _XTRA + _NBUF - 1) // _NBUF, ring_body, 0)

    def odrain(r, _):
        pltpu.make_async_copy(
            out_hbm.at[pl.ds(0, EMBED_DIM)],
            oring.at[pl.ds(0, EMBED_DIM)],
            osem,
        ).wait()
        return 0

    lax.fori_loop(0, jnp.minimum(off, _ORING), odrain, 0)


def kernel(user_inputs, table):
    tt = jnp.transpose(table)
    tail = table[_TAIL:, :]
    out_flat = _sc_gather(user_inputs, tt, tail)
    return jnp.reshape(out_flat, (BATCH, EMBED_DIM))
